# baseline (speedup 1.0000x reference)
import functools

import jax
import jax.numpy as jnp
from jax import lax
from jax.experimental import pallas as pl
from jax.experimental.pallas import tpu as pltpu

_DELTA_VAR = 0.5
_DELTA_DIST = 1.5
_ALPHA = 0.1
_BETA = 1.0
_GAMMA = 0.001
_C = 64


def _loss_body(feat_ref, lab_ref, out_ref, sums, counts, hc, means_b, nmrow,
               cache, *, nb, nc, rows):
    p = pl.program_id(0)
    j = pl.program_id(1)

    labels = lab_ref[0, 0, :]
    r = rows
    cls = lax.broadcasted_iota(jnp.int32, (r, _C), 1)
    oh_f = (labels[:, None] == cls).astype(jnp.float32)
    oh_b = oh_f.astype(jnp.bfloat16)

    @pl.when((p == 0) & (j == 0))
    def _init():
        sums[...] = jnp.zeros_like(sums)
        counts[...] = jnp.zeros_like(counts)
        hc[...] = jnp.zeros_like(hc)

    @pl.when(p == 0)
    def _pass0():
        fb = feat_ref[...].astype(jnp.bfloat16)
        sums[...] += lax.dot_general(
            oh_b, fb, (((0,), (0,)), ((), ())),
            preferred_element_type=jnp.float32)
        counts[...] += lax.dot_general(
            oh_b, jnp.ones((r, 8), jnp.bfloat16), (((0,), (0,)), ((), ())),
            preferred_element_type=jnp.float32)[:, 0:1]

        @pl.when(j < nc)
        def _fill_cache():
            cache[pl.ds(j * r, r), :] = fb

    @pl.when((p == 1) & (j == 0))
    def _make_means():
        m = sums[...] / jnp.maximum(counts[...], 1.0)
        means_b[...] = m.astype(jnp.bfloat16)
        nmrow[...] = lax.dot_general(
            jnp.ones((1, m.shape[1]), jnp.float32), m * m,
            (((1,), (1,)), ((), ())),
            preferred_element_type=jnp.float32)

    def _hinge_accum(fb):
        g = lax.dot_general(fb, means_b[...], (((1,), (1,)), ((), ())),
                            preferred_element_type=jnp.float32)
        f2 = fb * fb
        q = lax.dot_general(f2, jnp.ones((8, f2.shape[1]), jnp.bfloat16),
                            (((1,), (1,)), ((), ())),
                            preferred_element_type=jnp.float32)[:, 0:1]
        w = (2.0 * g - nmrow[...]) * oh_f
        s = jnp.sum(w, axis=1, keepdims=True)
        d2 = q - s
        dist = jnp.sqrt(jnp.maximum(d2, 1e-12))
        h = jnp.maximum(dist - _DELTA_VAR, 0.0)
        h2 = h * h
        hc[...] += lax.dot_general(
            oh_f, h2, (((0,), (0,)), ((), ())),
            preferred_element_type=jnp.float32)

    @pl.when((p == 1) & (j < nc))
    def _pass1_cached():
        _hinge_accum(cache[pl.ds(j * r, r), :])

    @pl.when((p == 1) & (j >= nc))
    def _pass1_hbm():
        _hinge_accum(feat_ref[...].astype(jnp.bfloat16))

    @pl.when((p == 1) & (j == nb - 1))
    def _finish():
        m = means_b[...].astype(jnp.float32)
        safe = jnp.maximum(counts[...], 1.0)
        var_loss = jnp.sum(hc[...] / safe) / float(_C)

        nm_col = lax.dot_general(
            m * m, jnp.ones((1, m.shape[1]), jnp.float32),
            (((1,), (1,)), ((), ())),
            preferred_element_type=jnp.float32)
        mmt = lax.dot_general(
            m, m, (((1,), (1,)), ((), ())),
            preferred_element_type=jnp.float32)
        sq = nm_col + nmrow[...] - 2.0 * mmt
        rr = lax.broadcasted_iota(jnp.int32, (_C, _C), 0)
        cc = lax.broadcasted_iota(jnp.int32, (_C, _C), 1)
        eye = rr == cc
        dist = jnp.sqrt(jnp.where(eye, 1.0, sq))
        hh = jnp.maximum(2.0 * _DELTA_DIST - dist, 0.0)
        hh = hh * hh
        hh = jnp.where(eye, 0.0, hh)
        dist_loss = jnp.sum(hh) / float((_C - 1) * _C)

        reg_loss = jnp.sum(jnp.sqrt(jnp.maximum(nm_col, 1e-12))) / float(_C)

        total = _ALPHA * var_loss + _BETA * dist_loss + _GAMMA * reg_loss
        out_ref[...] = jnp.reshape(total, (1, 1))


def kernel(features, labels):
    n, d = features.shape
    rows = 2000
    nb = n // rows
    nc = 54
    labels3 = labels.astype(jnp.int32).reshape(nb, 1, rows)

    out = pl.pallas_call(
        functools.partial(_loss_body, nb=nb, nc=nc, rows=rows),
        grid=(2, nb),
        in_specs=[
            pl.BlockSpec((rows, d),
                         lambda p, j: (jnp.where((p == 1) & (j < nc),
                                                 nb - 1, j), 0)),
            pl.BlockSpec((1, 1, rows), lambda p, j: (j, 0, 0)),
        ],
        out_specs=pl.BlockSpec((1, 1), lambda p, j: (0, 0)),
        out_shape=jax.ShapeDtypeStruct((1, 1), jnp.float32),
        scratch_shapes=[
            pltpu.VMEM((_C, d), jnp.float32),
            pltpu.VMEM((_C, 1), jnp.float32),
            pltpu.VMEM((_C, 1), jnp.float32),
            pltpu.VMEM((_C, d), jnp.bfloat16),
            pltpu.VMEM((1, _C), jnp.float32),
            pltpu.VMEM((54 * 2000, d), jnp.bfloat16),
        ],
        compiler_params=pltpu.CompilerParams(
            dimension_semantics=("arbitrary", "arbitrary"),
            vmem_limit_bytes=64 * 1024 * 1024,
        ),
    )(features, labels3)
    return out[0, 0]


# speedup vs baseline: 1.2266x; 1.2266x over previous
import functools

import jax
import jax.numpy as jnp
from jax import lax
from jax.experimental import pallas as pl
from jax.experimental.pallas import tpu as pltpu

_DELTA_VAR = 0.5
_DELTA_DIST = 1.5
_ALPHA = 0.1
_BETA = 1.0
_GAMMA = 0.001
_C = 64


def _loss_body(feat_ref, lab_ref, out_ref, sums, counts, hc, means_b, nmrow,
               cache, *, nb, nc, rows):
    p = pl.program_id(0)
    j = pl.program_id(1)

    labels = lab_ref[0, 0, :]
    r = rows
    cls = lax.broadcasted_iota(jnp.int32, (r, _C), 1)
    oh_f = (labels[:, None] == cls).astype(jnp.float32)
    oh_b = oh_f.astype(jnp.bfloat16)

    @pl.when((p == 0) & (j == 0))
    def _init():
        sums[...] = jnp.zeros_like(sums)
        counts[...] = jnp.zeros_like(counts)
        hc[...] = jnp.zeros_like(hc)

    @pl.when(p == 0)
    def _pass0():
        fb = feat_ref[...].astype(jnp.bfloat16)
        sums[...] += lax.dot_general(
            oh_b, fb, (((0,), (0,)), ((), ())),
            preferred_element_type=jnp.float32)
        counts[...] += lax.dot_general(
            oh_b, jnp.ones((r, 8), jnp.bfloat16), (((0,), (0,)), ((), ())),
            preferred_element_type=jnp.float32)[:, 0:1]

        @pl.when(j < nc)
        def _fill_cache():
            cache[pl.ds(j * r, r), :] = fb

    @pl.when((p == 1) & (j == 0))
    def _make_means():
        m = sums[...] / jnp.maximum(counts[...], 1.0)
        means_b[...] = m.astype(jnp.bfloat16)
        nmrow[...] = lax.dot_general(
            jnp.ones((1, m.shape[1]), jnp.float32), m * m,
            (((1,), (1,)), ((), ())),
            preferred_element_type=jnp.float32)

    def _hinge_accum(fb):
        g = lax.dot_general(fb, means_b[...], (((1,), (1,)), ((), ())),
                            preferred_element_type=jnp.float32)
        f2 = fb * fb
        q = lax.dot_general(f2, jnp.ones((8, f2.shape[1]), jnp.bfloat16),
                            (((1,), (1,)), ((), ())),
                            preferred_element_type=jnp.float32)[:, 0:1]
        w = (2.0 * g - nmrow[...]) * oh_f
        s = jnp.sum(w, axis=1, keepdims=True)
        d2 = q - s
        dist = jnp.sqrt(jnp.maximum(d2, 1e-12))
        h = jnp.maximum(dist - _DELTA_VAR, 0.0)
        h2 = h * h
        hc[...] += lax.dot_general(
            oh_f, h2, (((0,), (0,)), ((), ())),
            preferred_element_type=jnp.float32)

    @pl.when((p == 1) & (j < nc))
    def _pass1_cached():
        _hinge_accum(cache[pl.ds(j * r, r), :])

    @pl.when((p == 1) & (j >= nc))
    def _pass1_hbm():
        _hinge_accum(feat_ref[...].astype(jnp.bfloat16))

    @pl.when((p == 1) & (j == nb - 1))
    def _finish():
        m = means_b[...].astype(jnp.float32)
        safe = jnp.maximum(counts[...], 1.0)
        var_loss = jnp.sum(hc[...] / safe) / float(_C)

        nm_col = lax.dot_general(
            m * m, jnp.ones((1, m.shape[1]), jnp.float32),
            (((1,), (1,)), ((), ())),
            preferred_element_type=jnp.float32)
        mmt = lax.dot_general(
            m, m, (((1,), (1,)), ((), ())),
            preferred_element_type=jnp.float32)
        sq = nm_col + nmrow[...] - 2.0 * mmt
        rr = lax.broadcasted_iota(jnp.int32, (_C, _C), 0)
        cc = lax.broadcasted_iota(jnp.int32, (_C, _C), 1)
        eye = rr == cc
        dist = jnp.sqrt(jnp.where(eye, 1.0, sq))
        hh = jnp.maximum(2.0 * _DELTA_DIST - dist, 0.0)
        hh = hh * hh
        hh = jnp.where(eye, 0.0, hh)
        dist_loss = jnp.sum(hh) / float((_C - 1) * _C)

        reg_loss = jnp.sum(jnp.sqrt(jnp.maximum(nm_col, 1e-12))) / float(_C)

        total = _ALPHA * var_loss + _BETA * dist_loss + _GAMMA * reg_loss
        out_ref[...] = jnp.reshape(total, (1, 1))


def kernel(features, labels):
    n, d = features.shape
    rows = 4000
    nb = n // rows
    nc = 25
    labels3 = labels.astype(jnp.int32).reshape(nb, 1, rows)

    out = pl.pallas_call(
        functools.partial(_loss_body, nb=nb, nc=nc, rows=rows),
        grid=(2, nb),
        in_specs=[
            pl.BlockSpec((rows, d),
                         lambda p, j: (jnp.where((p == 1) & (j < nc),
                                                 nb - 1, j), 0)),
            pl.BlockSpec((1, 1, rows), lambda p, j: (j, 0, 0)),
        ],
        out_specs=pl.BlockSpec((1, 1), lambda p, j: (0, 0)),
        out_shape=jax.ShapeDtypeStruct((1, 1), jnp.float32),
        scratch_shapes=[
            pltpu.VMEM((_C, d), jnp.float32),
            pltpu.VMEM((_C, 1), jnp.float32),
            pltpu.VMEM((_C, 1), jnp.float32),
            pltpu.VMEM((_C, d), jnp.bfloat16),
            pltpu.VMEM((1, _C), jnp.float32),
            pltpu.VMEM((25 * 4000, d), jnp.bfloat16),
        ],
        compiler_params=pltpu.CompilerParams(
            dimension_semantics=("arbitrary", "arbitrary"),
            vmem_limit_bytes=64 * 1024 * 1024,
        ),
    )(features, labels3)
    return out[0, 0]


# speedup vs baseline: 1.7094x; 1.3937x over previous
import functools

import jax
import jax.numpy as jnp
from jax import lax
from jax.experimental import pallas as pl
from jax.experimental.pallas import tpu as pltpu

_DELTA_VAR = 0.5
_DELTA_DIST = 1.5
_ALPHA = 0.1
_BETA = 1.0
_GAMMA = 0.001
_C = 64


def _loss_body(feat_ref, lab_ref, out_ref, sums, counts, hc, means_b, nmcol,
               qn, cache, *, nb, nc, rows):
    p = pl.program_id(0)
    j = pl.program_id(1)

    labels2 = lab_ref[0]
    r = rows
    d = feat_ref.shape[1]
    cls = lax.broadcasted_iota(jnp.int32, (_C, r), 0)
    oh_f = (labels2 == cls).astype(jnp.float32)
    oh_b = oh_f.astype(jnp.bfloat16)

    @pl.when((p == 0) & (j == 0))
    def _init():
        sums[...] = jnp.zeros_like(sums)
        counts[...] = jnp.zeros_like(counts)
        hc[...] = jnp.zeros_like(hc)

    @pl.when(p == 0)
    def _pass0():
        fb = feat_ref[...].astype(jnp.bfloat16)
        sums[...] += lax.dot_general(
            oh_b, fb, (((1,), (0,)), ((), ())),
            preferred_element_type=jnp.float32)
        counts[...] += lax.dot_general(
            oh_b, jnp.ones((8, r), jnp.bfloat16), (((1,), (1,)), ((), ())),
            preferred_element_type=jnp.float32)[:, 0:1]
        f2 = fb * fb
        q8 = lax.dot_general(
            jnp.ones((8, d), jnp.bfloat16), f2, (((1,), (1,)), ((), ())),
            preferred_element_type=jnp.float32)
        qn[pl.ds(j, 1), :] = q8[0:1, :]

        @pl.when(j < nc)
        def _fill_cache():
            cache[pl.ds(j * r, r), :] = fb

    @pl.when((p == 1) & (j == 0))
    def _make_means():
        m = sums[...] / jnp.maximum(counts[...], 1.0)
        means_b[...] = m.astype(jnp.bfloat16)
        nmcol[...] = lax.dot_general(
            m * m, jnp.ones((1, d), jnp.float32), (((1,), (1,)), ((), ())),
            preferred_element_type=jnp.float32)

    def _hinge_accum(fb):
        gt = lax.dot_general(
            means_b[...], fb, (((1,), (1,)), ((), ())),
            preferred_element_type=jnp.float32)
        w = (2.0 * gt - nmcol[...]) * oh_f
        s = jnp.sum(w, axis=0, keepdims=True)
        d2 = qn[pl.ds(j, 1), :] - s
        dist = jnp.sqrt(jnp.maximum(d2, 1e-12))
        h = jnp.maximum(dist - _DELTA_VAR, 0.0)
        h2 = h * h
        hc[...] += lax.dot_general(
            oh_f, h2, (((1,), (1,)), ((), ())),
            preferred_element_type=jnp.float32)

    @pl.when((p == 1) & (j < nc))
    def _pass1_cached():
        _hinge_accum(cache[pl.ds(j * r, r), :])

    @pl.when((p == 1) & (j >= nc))
    def _pass1_hbm():
        _hinge_accum(feat_ref[...].astype(jnp.bfloat16))

    @pl.when((p == 1) & (j == nb - 1))
    def _finish():
        m = means_b[...].astype(jnp.float32)
        safe = jnp.maximum(counts[...], 1.0)
        var_loss = jnp.sum(hc[...] / safe) / float(_C)

        nm_row = lax.dot_general(
            jnp.ones((1, d), jnp.float32), m * m, (((1,), (1,)), ((), ())),
            preferred_element_type=jnp.float32)
        mmt = lax.dot_general(
            m, m, (((1,), (1,)), ((), ())),
            preferred_element_type=jnp.float32)
        sq = nmcol[...] + nm_row - 2.0 * mmt
        rr = lax.broadcasted_iota(jnp.int32, (_C, _C), 0)
        cc = lax.broadcasted_iota(jnp.int32, (_C, _C), 1)
        eye = rr == cc
        dist = jnp.sqrt(jnp.where(eye, 1.0, sq))
        hh = jnp.maximum(2.0 * _DELTA_DIST - dist, 0.0)
        hh = hh * hh
        hh = jnp.where(eye, 0.0, hh)
        dist_loss = jnp.sum(hh) / float((_C - 1) * _C)

        reg_loss = jnp.sum(jnp.sqrt(jnp.maximum(nmcol[...], 1e-12))) / float(_C)

        total = _ALPHA * var_loss + _BETA * dist_loss + _GAMMA * reg_loss
        out_ref[...] = jnp.reshape(total, (1, 1))


def kernel(features, labels):
    n, d = features.shape
    rows = 4000
    nb = n // rows
    nc = 25
    labels3 = labels.astype(jnp.int32).reshape(nb, 1, rows)

    out = pl.pallas_call(
        functools.partial(_loss_body, nb=nb, nc=nc, rows=rows),
        grid=(2, nb),
        in_specs=[
            pl.BlockSpec((rows, d),
                         lambda p, j: (jnp.where((p == 1) & (j < nc),
                                                 nb - 1, j), 0)),
            pl.BlockSpec((1, 1, rows), lambda p, j: (j, 0, 0)),
        ],
        out_specs=pl.BlockSpec((1, 1), lambda p, j: (0, 0)),
        out_shape=jax.ShapeDtypeStruct((1, 1), jnp.float32),
        scratch_shapes=[
            pltpu.VMEM((_C, d), jnp.float32),
            pltpu.VMEM((_C, 1), jnp.float32),
            pltpu.VMEM((_C, 1), jnp.float32),
            pltpu.VMEM((_C, d), jnp.bfloat16),
            pltpu.VMEM((_C, 1), jnp.float32),
            pltpu.VMEM((40, 4000), jnp.float32),
            pltpu.VMEM((25 * 4000, d), jnp.bfloat16),
        ],
        compiler_params=pltpu.CompilerParams(
            dimension_semantics=("arbitrary", "arbitrary"),
            vmem_limit_bytes=64 * 1024 * 1024,
        ),
    )(features, labels3)
    return out[0, 0]


# speedup vs baseline: 1.8769x; 1.0980x over previous
import functools

import jax
import jax.numpy as jnp
from jax import lax
from jax.experimental import pallas as pl
from jax.experimental.pallas import tpu as pltpu

_DELTA_VAR = 0.5
_DELTA_DIST = 1.5
_ALPHA = 0.1
_BETA = 1.0
_GAMMA = 0.001
_C = 64


def _loss_body(feat_ref, lab_ref, out_ref, sums, counts, hc, means_b, nmcol,
               qn, cache, *, nb, rows):
    p = pl.program_id(0)
    j = pl.program_id(1)

    labels2 = lab_ref[0]
    r = rows
    d = feat_ref.shape[1]
    cls = lax.broadcasted_iota(jnp.int32, (_C, r), 0)
    oh_f = (labels2 == cls).astype(jnp.float32)
    oh_b = oh_f.astype(jnp.bfloat16)

    @pl.when((p == 0) & (j == 0))
    def _init():
        sums[...] = jnp.zeros_like(sums)
        counts[...] = jnp.zeros_like(counts)
        hc[...] = jnp.zeros_like(hc)

    @pl.when(p == 0)
    def _pass0():
        fb = feat_ref[...].astype(jnp.bfloat16)
        sums[...] += lax.dot_general(
            oh_b, fb, (((1,), (0,)), ((), ())),
            preferred_element_type=jnp.float32)
        counts[...] += lax.dot_general(
            oh_b, jnp.ones((8, r), jnp.bfloat16), (((1,), (1,)), ((), ())),
            preferred_element_type=jnp.float32)[:, 0:1]
        f2 = fb * fb
        q8 = lax.dot_general(
            jnp.ones((8, d), jnp.bfloat16), f2, (((1,), (1,)), ((), ())),
            preferred_element_type=jnp.float32)
        qn[pl.ds(j, 1), :] = q8[0:1, :]
        cache[pl.ds(j * r, r), :] = fb.astype(jnp.float8_e4m3fn)

    @pl.when((p == 1) & (j == 0))
    def _make_means():
        m = sums[...] / jnp.maximum(counts[...], 1.0)
        means_b[...] = m.astype(jnp.bfloat16)
        nmcol[...] = lax.dot_general(
            m * m, jnp.ones((1, d), jnp.float32), (((1,), (1,)), ((), ())),
            preferred_element_type=jnp.float32)

    def _hinge_accum(fb):
        gt = lax.dot_general(
            means_b[...], fb, (((1,), (1,)), ((), ())),
            preferred_element_type=jnp.float32)
        w = (2.0 * gt - nmcol[...]) * oh_f
        s = jnp.sum(w, axis=0, keepdims=True)
        d2 = qn[pl.ds(j, 1), :] - s
        dist = jnp.sqrt(jnp.maximum(d2, 1e-12))
        h = jnp.maximum(dist - _DELTA_VAR, 0.0)
        h2 = h * h
        hc[...] += lax.dot_general(
            oh_f, h2, (((1,), (1,)), ((), ())),
            preferred_element_type=jnp.float32)

    @pl.when(p == 1)
    def _pass1():
        _hinge_accum(cache[pl.ds(j * r, r), :].astype(jnp.bfloat16))

    @pl.when((p == 1) & (j == nb - 1))
    def _finish():
        m = means_b[...].astype(jnp.float32)
        safe = jnp.maximum(counts[...], 1.0)
        var_loss = jnp.sum(hc[...] / safe) / float(_C)

        nm_row = lax.dot_general(
            jnp.ones((1, d), jnp.float32), m * m, (((1,), (1,)), ((), ())),
            preferred_element_type=jnp.float32)
        mmt = lax.dot_general(
            m, m, (((1,), (1,)), ((), ())),
            preferred_element_type=jnp.float32)
        sq = nmcol[...] + nm_row - 2.0 * mmt
        rr = lax.broadcasted_iota(jnp.int32, (_C, _C), 0)
        cc = lax.broadcasted_iota(jnp.int32, (_C, _C), 1)
        eye = rr == cc
        dist = jnp.sqrt(jnp.where(eye, 1.0, sq))
        hh = jnp.maximum(2.0 * _DELTA_DIST - dist, 0.0)
        hh = hh * hh
        hh = jnp.where(eye, 0.0, hh)
        dist_loss = jnp.sum(hh) / float((_C - 1) * _C)

        reg_loss = jnp.sum(jnp.sqrt(jnp.maximum(nmcol[...], 1e-12))) / float(_C)

        total = _ALPHA * var_loss + _BETA * dist_loss + _GAMMA * reg_loss
        out_ref[...] = jnp.reshape(total, (1, 1))


def kernel(features, labels):
    n, d = features.shape
    rows = 4000
    nb = n // rows
    labels3 = labels.astype(jnp.int32).reshape(nb, 1, rows)

    out = pl.pallas_call(
        functools.partial(_loss_body, nb=nb, rows=rows),
        grid=(2, nb),
        in_specs=[
            pl.BlockSpec((rows, d),
                         lambda p, j: (jnp.where(p == 1, nb - 1, j), 0)),
            pl.BlockSpec((1, 1, rows), lambda p, j: (j, 0, 0)),
        ],
        out_specs=pl.BlockSpec((1, 1), lambda p, j: (0, 0)),
        out_shape=jax.ShapeDtypeStruct((1, 1), jnp.float32),
        scratch_shapes=[
            pltpu.VMEM((_C, d), jnp.float32),
            pltpu.VMEM((_C, 1), jnp.float32),
            pltpu.VMEM((_C, 1), jnp.float32),
            pltpu.VMEM((_C, d), jnp.bfloat16),
            pltpu.VMEM((_C, 1), jnp.float32),
            pltpu.VMEM((40, 4000), jnp.float32),
            pltpu.VMEM((n, d), jnp.float8_e4m3fn),
        ],
        compiler_params=pltpu.CompilerParams(
            dimension_semantics=("arbitrary", "arbitrary"),
            vmem_limit_bytes=64 * 1024 * 1024,
        ),
    )(features, labels3)
    return out[0, 0]


# speedup vs baseline: 2.0967x; 1.1171x over previous
import functools

import jax
import jax.numpy as jnp
from jax import lax
from jax.experimental import pallas as pl
from jax.experimental.pallas import tpu as pltpu

_DELTA_VAR = 0.5
_DELTA_DIST = 1.5
_ALPHA = 0.1
_BETA = 1.0
_GAMMA = 0.001
_C = 64


def _loss_body(feat_ref, lab_ref, out_ref, sums, counts, hc, means_b, nmcol,
               qn, cache, *, nb, rows):
    p = pl.program_id(0)
    j = pl.program_id(1)

    labels2 = lab_ref[0]
    r = rows
    d = feat_ref.shape[1]
    cls = lax.broadcasted_iota(jnp.int32, (_C, r), 0)
    oh_f = (labels2 == cls).astype(jnp.float32)
    oh_b = oh_f.astype(jnp.bfloat16)

    @pl.when((p == 0) & (j == 0))
    def _init():
        sums[...] = jnp.zeros_like(sums)
        counts[...] = jnp.zeros_like(counts)
        hc[...] = jnp.zeros_like(hc)

    @pl.when(p == 0)
    def _pass0():
        fb = feat_ref[...].astype(jnp.bfloat16)
        sums[...] += lax.dot_general(
            oh_b, fb, (((1,), (0,)), ((), ())),
            preferred_element_type=jnp.float32)
        counts[...] += lax.dot_general(
            oh_b, jnp.ones((8, r), jnp.bfloat16), (((1,), (1,)), ((), ())),
            preferred_element_type=jnp.float32)[:, 0:1]
        f2 = fb * fb
        q8 = lax.dot_general(
            jnp.ones((8, d), jnp.bfloat16), f2, (((1,), (1,)), ((), ())),
            preferred_element_type=jnp.float32)
        qn[pl.ds(j, 1), :] = q8[0:1, :]
        cache[pl.ds(j * r, r), :] = fb.astype(jnp.float8_e4m3fn)

    @pl.when((p == 1) & (j == 0))
    def _make_means():
        m = sums[...] / jnp.maximum(counts[...], 1.0)
        means_b[...] = m.astype(jnp.float8_e4m3fn)
        nmcol[...] = lax.dot_general(
            m * m, jnp.ones((1, d), jnp.float32), (((1,), (1,)), ((), ())),
            preferred_element_type=jnp.float32)

    def _hinge_accum(fb):
        gt = lax.dot_general(
            means_b[...], fb, (((1,), (1,)), ((), ())),
            preferred_element_type=jnp.float32)
        w = (2.0 * gt - nmcol[...]) * oh_f
        s = jnp.sum(w, axis=0, keepdims=True)
        d2 = qn[pl.ds(j, 1), :] - s
        dist = jnp.sqrt(jnp.maximum(d2, 1e-12))
        h = jnp.maximum(dist - _DELTA_VAR, 0.0)
        h2 = h * h
        hc[...] += lax.dot_general(
            oh_f, h2, (((1,), (1,)), ((), ())),
            preferred_element_type=jnp.float32)

    @pl.when(p == 1)
    def _pass1():
        _hinge_accum(cache[pl.ds(j * r, r), :])

    @pl.when((p == 1) & (j == nb - 1))
    def _finish():
        safe = jnp.maximum(counts[...], 1.0)
        m = sums[...] / safe
        var_loss = jnp.sum(hc[...] / safe) / float(_C)

        nm_row = lax.dot_general(
            jnp.ones((1, d), jnp.float32), m * m, (((1,), (1,)), ((), ())),
            preferred_element_type=jnp.float32)
        mmt = lax.dot_general(
            m, m, (((1,), (1,)), ((), ())),
            preferred_element_type=jnp.float32)
        sq = nmcol[...] + nm_row - 2.0 * mmt
        rr = lax.broadcasted_iota(jnp.int32, (_C, _C), 0)
        cc = lax.broadcasted_iota(jnp.int32, (_C, _C), 1)
        eye = rr == cc
        dist = jnp.sqrt(jnp.where(eye, 1.0, sq))
        hh = jnp.maximum(2.0 * _DELTA_DIST - dist, 0.0)
        hh = hh * hh
        hh = jnp.where(eye, 0.0, hh)
        dist_loss = jnp.sum(hh) / float((_C - 1) * _C)

        reg_loss = jnp.sum(jnp.sqrt(jnp.maximum(nmcol[...], 1e-12))) / float(_C)

        total = _ALPHA * var_loss + _BETA * dist_loss + _GAMMA * reg_loss
        out_ref[...] = jnp.reshape(total, (1, 1))


def kernel(features, labels):
    n, d = features.shape
    rows = 4000
    nb = n // rows
    labels3 = labels.astype(jnp.int32).reshape(nb, 1, rows)

    out = pl.pallas_call(
        functools.partial(_loss_body, nb=nb, rows=rows),
        grid=(2, nb),
        in_specs=[
            pl.BlockSpec((rows, d),
                         lambda p, j: (jnp.where(p == 1, nb - 1, j), 0)),
            pl.BlockSpec((1, 1, rows), lambda p, j: (j, 0, 0)),
        ],
        out_specs=pl.BlockSpec((1, 1), lambda p, j: (0, 0)),
        out_shape=jax.ShapeDtypeStruct((1, 1), jnp.float32),
        scratch_shapes=[
            pltpu.VMEM((_C, d), jnp.float32),
            pltpu.VMEM((_C, 1), jnp.float32),
            pltpu.VMEM((_C, 1), jnp.float32),
            pltpu.VMEM((_C, d), jnp.float8_e4m3fn),
            pltpu.VMEM((_C, 1), jnp.float32),
            pltpu.VMEM((40, 4000), jnp.float32),
            pltpu.VMEM((n, d), jnp.float8_e4m3fn),
        ],
        compiler_params=pltpu.CompilerParams(
            dimension_semantics=("arbitrary", "arbitrary"),
            vmem_limit_bytes=64 * 1024 * 1024,
        ),
    )(features, labels3)
    return out[0, 0]


# speedup vs baseline: 2.5489x; 1.2157x over previous
import functools

import jax
import jax.numpy as jnp
from jax import lax
from jax.experimental import pallas as pl
from jax.experimental.pallas import tpu as pltpu

_DELTA_VAR = 0.5
_DELTA_DIST = 1.5
_ALPHA = 0.1
_BETA = 1.0
_GAMMA = 0.001
_C = 64


def _loss_body(feat_ref, lab_ref, out_ref, sums, counts, hc, means_b, nmcol,
               qn, cache, *, nb, rows):
    p = pl.program_id(0)
    j = pl.program_id(1)

    labels2 = lab_ref[0]
    r = rows
    d = feat_ref.shape[1]
    cls = lax.broadcasted_iota(jnp.int32, (_C, r), 0)
    oh_f = (labels2 == cls).astype(jnp.float32)
    oh_b = oh_f.astype(jnp.bfloat16)

    @pl.when((p == 0) & (j == 0))
    def _init():
        sums[...] = jnp.zeros_like(sums)
        counts[...] = jnp.zeros_like(counts)
        hc[...] = jnp.zeros_like(hc)

    @pl.when(p == 0)
    def _pass0():
        fb = feat_ref[...].astype(jnp.bfloat16)
        sums[...] += lax.dot_general(
            oh_b, fb, (((1,), (0,)), ((), ())),
            preferred_element_type=jnp.float32)
        counts[...] += lax.dot_general(
            oh_b, jnp.ones((8, r), jnp.bfloat16), (((1,), (1,)), ((), ())),
            preferred_element_type=jnp.float32)[:, 0:1]
        f2 = fb * fb
        q8 = lax.dot_general(
            jnp.ones((8, d), jnp.bfloat16), f2, (((1,), (1,)), ((), ())),
            preferred_element_type=jnp.float32)
        qn[pl.ds(j, 1), :] = q8[0:1, :]
        cache[pl.ds(j * r, r), :] = fb.astype(jnp.float8_e4m3fn)

    @pl.when((p == 1) & (j == 0))
    def _make_means():
        m = sums[...] / jnp.maximum(counts[...], 1.0)
        means_b[...] = m.astype(jnp.float8_e4m3fn)
        nmcol[...] = lax.dot_general(
            m * m, jnp.ones((1, d), jnp.float32), (((1,), (1,)), ((), ())),
            preferred_element_type=jnp.float32)

    def _hinge_accum(fb):
        gt = lax.dot_general(
            means_b[...], fb, (((1,), (1,)), ((), ())),
            preferred_element_type=jnp.float32)
        w = (2.0 * gt - nmcol[...]) * oh_f
        s = jnp.sum(w, axis=0, keepdims=True)
        d2 = qn[pl.ds(j, 1), :] - s
        dist = jnp.sqrt(jnp.maximum(d2, 1e-12))
        h = jnp.maximum(dist - _DELTA_VAR, 0.0)
        h2 = h * h
        hc[...] += lax.dot_general(
            oh_f, h2, (((1,), (1,)), ((), ())),
            preferred_element_type=jnp.float32)

    @pl.when(p == 1)
    def _pass1():
        _hinge_accum(cache[pl.ds(j * r, r), :])

    @pl.when((p == 1) & (j == nb - 1))
    def _finish():
        safe = jnp.maximum(counts[...], 1.0)
        m = sums[...] / safe
        var_loss = jnp.sum(hc[...] / safe) / float(_C)

        nm_row = lax.dot_general(
            jnp.ones((1, d), jnp.float32), m * m, (((1,), (1,)), ((), ())),
            preferred_element_type=jnp.float32)
        mmt = lax.dot_general(
            m, m, (((1,), (1,)), ((), ())),
            preferred_element_type=jnp.float32)
        sq = nmcol[...] + nm_row - 2.0 * mmt
        rr = lax.broadcasted_iota(jnp.int32, (_C, _C), 0)
        cc = lax.broadcasted_iota(jnp.int32, (_C, _C), 1)
        eye = rr == cc
        dist = jnp.sqrt(jnp.where(eye, 1.0, sq))
        hh = jnp.maximum(2.0 * _DELTA_DIST - dist, 0.0)
        hh = hh * hh
        hh = jnp.where(eye, 0.0, hh)
        dist_loss = jnp.sum(hh) / float((_C - 1) * _C)

        reg_loss = jnp.sum(jnp.sqrt(jnp.maximum(nmcol[...], 1e-12))) / float(_C)

        total = _ALPHA * var_loss + _BETA * dist_loss + _GAMMA * reg_loss
        out_ref[...] = jnp.reshape(total, (1, 1))


def kernel(features, labels):
    n, d = features.shape
    rows = 8000
    nb = n // rows
    labels3 = labels.astype(jnp.int32).reshape(nb, 1, rows)

    out = pl.pallas_call(
        functools.partial(_loss_body, nb=nb, rows=rows),
        grid=(2, nb),
        in_specs=[
            pl.BlockSpec((rows, d),
                         lambda p, j: (jnp.where(p == 1, nb - 1, j), 0)),
            pl.BlockSpec((1, 1, rows), lambda p, j: (j, 0, 0)),
        ],
        out_specs=pl.BlockSpec((1, 1), lambda p, j: (0, 0)),
        out_shape=jax.ShapeDtypeStruct((1, 1), jnp.float32),
        scratch_shapes=[
            pltpu.VMEM((_C, d), jnp.float32),
            pltpu.VMEM((_C, 1), jnp.float32),
            pltpu.VMEM((_C, 1), jnp.float32),
            pltpu.VMEM((_C, d), jnp.float8_e4m3fn),
            pltpu.VMEM((_C, 1), jnp.float32),
            pltpu.VMEM((20, 8000), jnp.float32),
            pltpu.VMEM((n, d), jnp.float8_e4m3fn),
        ],
        compiler_params=pltpu.CompilerParams(
            dimension_semantics=("arbitrary", "arbitrary"),
            vmem_limit_bytes=64 * 1024 * 1024,
        ),
    )(features, labels3)
    return out[0, 0]
